# baseline (speedup 1.0000x reference)
import functools

import jax
import jax.numpy as jnp
from jax import lax
from jax.experimental import pallas as pl
from jax.experimental.pallas import tpu as pltpu
from jax.experimental.pallas import tpu_sc as plsc

N_NODES = 10000
N_EDGES = 320000
D = 128
NR = 16

NC = 2
NS = 16
NW = NC * NS
EPW = N_EDGES // NW
CHUNK = 40
N_PAIRS = EPW // (2 * CHUNK)
EB = 4
LANES = 16
NCB = D // LANES

ACC_BLOCK = 8000
D2 = D // 2


def _precompute_body(x_ref, wrbf_ref, wedge_ref, b_ref, idxj_ref, idxi_ref,
                     t_ref, wr_ref, idx_ref):
    x = x_ref[...]
    t_ref[0:N_NODES, :] = jnp.dot(
        x, wedge_ref[0:D, :], preferred_element_type=jnp.float32
    ).astype(jnp.bfloat16)
    t_ref[N_NODES:2 * N_NODES, :] = (
        jnp.dot(x, wedge_ref[D:2 * D, :], preferred_element_type=jnp.float32)
        + b_ref[...]
    ).astype(jnp.bfloat16)
    wr_ref[...] = jnp.dot(wrbf_ref[...], wedge_ref[2 * D:3 * D, :],
                          preferred_element_type=jnp.float32)
    idx_ref[0] = idxj_ref[...]
    idx_ref[1] = idxi_ref[...] + N_NODES


def _precompute(x, W_rbf, W_edge, b_edge, idx_j, idx_i):
    return pl.pallas_call(
        _precompute_body,
        out_shape=[
            jax.ShapeDtypeStruct((2 * N_NODES, D), jnp.bfloat16),
            jax.ShapeDtypeStruct((NR, D), jnp.float32),
            jax.ShapeDtypeStruct((2, N_EDGES // D, D), jnp.int32),
        ],
    )(x, W_rbf, W_edge, b_edge.reshape(1, D),
      idx_j.reshape(N_EDGES // D, D), idx_i.reshape(N_EDGES // D, D))


def _acc_body(rbf_ref, wr_ref, acc_ref):
    acc_ref[...] = jnp.dot(rbf_ref[...], wr_ref[...],
                           preferred_element_type=jnp.float32
                           ).astype(jnp.bfloat16)


def _acc_matmul(rbf, wr):
    return pl.pallas_call(
        _acc_body,
        grid=(N_EDGES // ACC_BLOCK,),
        in_specs=[
            pl.BlockSpec((ACC_BLOCK, NR), lambda i: (i, 0)),
            pl.BlockSpec((NR, D), lambda i: (0, 0)),
        ],
        out_specs=pl.BlockSpec((ACC_BLOCK, D), lambda i: (i, 0)),
        out_shape=jax.ShapeDtypeStruct((N_EDGES, D), jnp.bfloat16),
    )(rbf, wr)


def _edge_body(t_hbm, acc_hbm, idx_hbm, out_hbm,
               idx_v0, idx_v1, rows_j, rows_i, acc_v, out_v,
               sem_g0, sem_g1, sem_a0, sem_a1, sem_o0, sem_o1):
    wid = lax.axis_index("s") * NC + lax.axis_index("c")
    base_w = wid * EPW
    sem_g = (sem_g0, sem_g1)
    sem_a = (sem_a0, sem_a1)
    sem_o = (sem_o0, sem_o1)
    rows = ((rows_j.at[0], rows_i.at[0]), (rows_j.at[1], rows_i.at[1]))
    accb = (acc_v.at[0], acc_v.at[1])
    outb = (out_v.at[0], out_v.at[1])

    pltpu.sync_copy(idx_hbm.at[0, wid, 0, :], idx_v0)
    pltpu.sync_copy(idx_hbm.at[1, wid, 0, :], idx_v1)

    def issue_in(c, b):
        off = c * CHUNK
        pltpu.async_copy(t_hbm.at[idx_v0.at[pl.ds(off, CHUNK)]],
                         rows[b][0], sem_g[b])
        pltpu.async_copy(t_hbm.at[idx_v1.at[pl.ds(off, CHUNK)]],
                         rows[b][1], sem_g[b])
        pltpu.async_copy(acc_hbm.at[pl.ds(base_w + off, CHUNK)],
                         accb[b], sem_a[b])

    def wait_in(b):
        pltpu.make_async_copy(t_hbm.at[idx_v0.at[pl.ds(0, CHUNK)]],
                              rows[b][0], sem_g[b]).wait()
        pltpu.make_async_copy(t_hbm.at[idx_v1.at[pl.ds(0, CHUNK)]],
                              rows[b][1], sem_g[b]).wait()
        pltpu.make_async_copy(acc_hbm.at[pl.ds(0, CHUNK)],
                              accb[b], sem_a[b]).wait()

    def wait_out(b):
        pltpu.make_async_copy(outb[b], out_hbm.at[pl.ds(0, CHUNK)],
                              sem_o[b]).wait()

    even_idx = [2 * lax.iota(jnp.int32, LANES) + 2 * LANES * g
                for g in range(D // (2 * LANES))]
    odd_idx = [ix + 1 for ix in even_idx]

    def compute_store(c, b):
        rj, ri = rows[b]
        av, ov = accb[b], outb[b]

        def eb_body(i, _):
            e0 = i * EB
            for ep in range(EB):
                e = e0 + ep
                esplat = jnp.full((LANES,), e, dtype=jnp.int32)
                for g in range(D // (2 * LANES)):
                    sl = pl.ds(LANES * g, LANES)
                    fmt = plsc.PackFormat.INTERLEAVED
                    aj, bj = plsc.unpack(
                        plsc.bitcast(rj[e, sl], jnp.bfloat16), format=fmt)
                    ai, bi = plsc.unpack(
                        plsc.bitcast(ri[e, sl], jnp.bfloat16), format=fmt)
                    aa, ba = plsc.unpack(
                        plsc.bitcast(av[e, sl], jnp.bfloat16), format=fmt)
                    ta = aj + ai + aa
                    tb = bj + bi + ba
                    plsc.store_scatter(ov, [esplat, even_idx[g]],
                                       ta / (1.0 + jnp.exp(-ta)))
                    plsc.store_scatter(ov, [esplat, odd_idx[g]],
                                       tb / (1.0 + jnp.exp(-tb)))
            return 0

        lax.fori_loop(0, CHUNK // EB, eb_body, 0)
        pltpu.async_copy(ov, out_hbm.at[pl.ds(base_w + c * CHUNK, CHUNK)],
                         sem_o[b])

    issue_in(0, 0)
    issue_in(1, 1)

    def pair_body(p, _):
        c0 = 2 * p
        for b in (0, 1):
            c = c0 + b
            wait_in(b)

            @pl.when(p > 0)
            def _():
                wait_out(b)

            compute_store(c, b)

            @pl.when(p < N_PAIRS - 1)
            def _():
                issue_in(c + 2, b)

        return 0

    lax.fori_loop(0, N_PAIRS, pair_body, 0)
    wait_out(0)
    wait_out(1)


def _edge_kernel(t, acc, idx_cat):
    mesh = plsc.VectorSubcoreMesh(core_axis_name="c", subcore_axis_name="s")
    return pl.kernel(
        _edge_body,
        out_type=jax.ShapeDtypeStruct((N_EDGES, D), jnp.float32),
        mesh=mesh,
        compiler_params=pltpu.CompilerParams(needs_layout_passes=False,
                                             use_tc_tiling_on_sc=False),
        scratch_types=[
            pltpu.VMEM((EPW,), jnp.int32),
            pltpu.VMEM((EPW,), jnp.int32),
            pltpu.VMEM((2, CHUNK, D2), jnp.int32),
            pltpu.VMEM((2, CHUNK, D2), jnp.int32),
            pltpu.VMEM((2, CHUNK, D2), jnp.int32),
            pltpu.VMEM((2, CHUNK, D), jnp.float32),
        ] + [pltpu.SemaphoreType.DMA] * 6,
    )(t, acc, idx_cat)


def _pack_i32(a_bf16):
    n, d = a_bf16.shape
    return lax.bitcast_convert_type(
        a_bf16.reshape(n, d // 2, 2), jnp.int32)


def kernel(x, rbf, idx_i, idx_j, W_rbf, W_edge, b_edge):
    idx_i = idx_i.astype(jnp.int32)
    idx_j = idx_j.astype(jnp.int32)
    t, wr, idx_cat = _precompute(x, W_rbf, W_edge, b_edge, idx_j, idx_i)
    acc = _acc_matmul(rbf, wr)
    return _edge_kernel(_pack_i32(t), _pack_i32(acc),
                        idx_cat.reshape(2, NW, 1, EPW))


# speedup vs baseline: 4.7409x; 4.7409x over previous
import functools

import jax
import jax.numpy as jnp
from jax import lax
from jax.experimental import pallas as pl
from jax.experimental.pallas import tpu as pltpu
from jax.experimental.pallas import tpu_sc as plsc

N_NODES = 10000
N_EDGES = 320000
D = 128
NR = 16

NC = 2
NS = 16
NW = NC * NS
EPW = N_EDGES // NW
CHUNK = 40
N_PAIRS = EPW // (2 * CHUNK)
EB = 4
LANES = 16
NCB = D // LANES

ACC_BLOCK = 8000


def _tc_body(x_ref, wrbf_ref, wedge_ref, b_ref, idxj_ref, idxi_ref, rbf_ref,
             t_ref, idx_ref, acc_ref):
    wr = jnp.dot(wrbf_ref[...], wedge_ref[2 * D:3 * D, :],
                 preferred_element_type=jnp.float32)
    acc_ref[...] = jnp.dot(rbf_ref[...], wr,
                           preferred_element_type=jnp.float32)

    @pl.when(pl.program_id(0) == 0)
    def _():
        x = x_ref[...]
        t_ref[0:N_NODES, :] = jnp.dot(x, wedge_ref[0:D, :],
                                      preferred_element_type=jnp.float32)
        t_ref[N_NODES:2 * N_NODES, :] = (
            jnp.dot(x, wedge_ref[D:2 * D, :],
                    preferred_element_type=jnp.float32)
            + b_ref[...]
        )
        idx_ref[0] = idxj_ref[...]
        idx_ref[1] = idxi_ref[...] + N_NODES


def _tc_stage(x, W_rbf, W_edge, b_edge, idx_j, idx_i, rbf):
    full = lambda shape: pl.BlockSpec(shape, lambda i: tuple(0 for _ in shape))
    n_idx_rows = N_EDGES // D
    return pl.pallas_call(
        _tc_body,
        grid=(N_EDGES // ACC_BLOCK,),
        in_specs=[
            full((N_NODES, D)),
            full((NR, D)),
            full((3 * D, D)),
            full((1, D)),
            full((n_idx_rows, D)),
            full((n_idx_rows, D)),
            pl.BlockSpec((ACC_BLOCK, NR), lambda i: (i, 0)),
        ],
        out_specs=[
            full((2 * N_NODES, D)),
            full((2, n_idx_rows, D)),
            pl.BlockSpec((ACC_BLOCK, D), lambda i: (i, 0)),
        ],
        out_shape=[
            jax.ShapeDtypeStruct((2 * N_NODES, D), jnp.float32),
            jax.ShapeDtypeStruct((2, n_idx_rows, D), jnp.int32),
            jax.ShapeDtypeStruct((N_EDGES, D), jnp.float32),
        ],
    )(x, W_rbf, W_edge, b_edge.reshape(1, D),
      idx_j.reshape(n_idx_rows, D), idx_i.reshape(n_idx_rows, D), rbf)


def _edge_body(t_hbm, acc_hbm, idx_hbm, out_hbm,
               idx_v0, idx_v1, rows_j, rows_i, acc_v, out_v,
               sem_g0, sem_g1, sem_a0, sem_a1, sem_o0, sem_o1):
    wid = lax.axis_index("s") * NC + lax.axis_index("c")
    base_w = wid * EPW
    sem_g = (sem_g0, sem_g1)
    sem_a = (sem_a0, sem_a1)
    sem_o = (sem_o0, sem_o1)
    rows = ((rows_j.at[0], rows_i.at[0]), (rows_j.at[1], rows_i.at[1]))
    accb = (acc_v.at[0], acc_v.at[1])
    outb = (out_v.at[0], out_v.at[1])

    pltpu.sync_copy(idx_hbm.at[0, wid, 0, :], idx_v0)
    pltpu.sync_copy(idx_hbm.at[1, wid, 0, :], idx_v1)

    def issue_in(c, b):
        off = c * CHUNK
        pltpu.async_copy(t_hbm.at[idx_v0.at[pl.ds(off, CHUNK)]],
                         rows[b][0], sem_g[b])
        pltpu.async_copy(t_hbm.at[idx_v1.at[pl.ds(off, CHUNK)]],
                         rows[b][1], sem_g[b])
        pltpu.async_copy(acc_hbm.at[pl.ds(base_w + off, CHUNK)],
                         accb[b], sem_a[b])

    def wait_in(b):
        pltpu.make_async_copy(t_hbm.at[idx_v0.at[pl.ds(0, CHUNK)]],
                              rows[b][0], sem_g[b]).wait()
        pltpu.make_async_copy(t_hbm.at[idx_v1.at[pl.ds(0, CHUNK)]],
                              rows[b][1], sem_g[b]).wait()
        pltpu.make_async_copy(acc_hbm.at[pl.ds(0, CHUNK)],
                              accb[b], sem_a[b]).wait()

    def wait_out(b):
        pltpu.make_async_copy(outb[b], out_hbm.at[pl.ds(0, CHUNK)],
                              sem_o[b]).wait()

    def compute_store(c, b):
        rj, ri = rows[b]
        av, ov = accb[b], outb[b]

        def eb_body(i, _):
            e0 = i * EB
            for ep in range(EB):
                e = e0 + ep
                for cb in range(NCB):
                    sl = pl.ds(cb * LANES, LANES)
                    t = rj[e, sl] + ri[e, sl] + av[e, sl]
                    ov[e, sl] = t / (1.0 + jnp.exp(-t))
            return 0

        lax.fori_loop(0, CHUNK // EB, eb_body, 0)
        pltpu.async_copy(ov, out_hbm.at[pl.ds(base_w + c * CHUNK, CHUNK)],
                         sem_o[b])

    issue_in(0, 0)
    issue_in(1, 1)

    def pair_body(p, _):
        c0 = 2 * p
        for b in (0, 1):
            c = c0 + b
            wait_in(b)

            @pl.when(p > 0)
            def _():
                wait_out(b)

            compute_store(c, b)

            @pl.when(p < N_PAIRS - 1)
            def _():
                issue_in(c + 2, b)

        return 0

    lax.fori_loop(0, N_PAIRS, pair_body, 0)
    wait_out(0)
    wait_out(1)


def _edge_kernel(t, acc, idx_cat):
    mesh = plsc.VectorSubcoreMesh(core_axis_name="c", subcore_axis_name="s")
    return pl.kernel(
        _edge_body,
        out_type=jax.ShapeDtypeStruct((N_EDGES, D), jnp.float32),
        mesh=mesh,
        scratch_types=[
            pltpu.VMEM((EPW,), jnp.int32),
            pltpu.VMEM((EPW,), jnp.int32),
            pltpu.VMEM((2, CHUNK, D), jnp.float32),
            pltpu.VMEM((2, CHUNK, D), jnp.float32),
            pltpu.VMEM((2, CHUNK, D), jnp.float32),
            pltpu.VMEM((2, CHUNK, D), jnp.float32),
        ] + [pltpu.SemaphoreType.DMA] * 6,
    )(t, acc, idx_cat)


def kernel(x, rbf, idx_i, idx_j, W_rbf, W_edge, b_edge):
    idx_i = idx_i.astype(jnp.int32)
    idx_j = idx_j.astype(jnp.int32)
    t, idx_cat, acc = _tc_stage(x, W_rbf, W_edge, b_edge, idx_j, idx_i, rbf)
    return _edge_kernel(t, acc, idx_cat.reshape(2, NW, 1, EPW))


# speedup vs baseline: 5.2182x; 1.1007x over previous
import functools

import jax
import jax.numpy as jnp
from jax import lax
from jax.experimental import pallas as pl
from jax.experimental.pallas import tpu as pltpu
from jax.experimental.pallas import tpu_sc as plsc

N_NODES = 10000
N_EDGES = 320000
D = 128
NR = 16

NC = 2
NS = 16
NW = NC * NS
EPW = N_EDGES // NW
CHUNK = 80
N_CHUNKS = EPW // CHUNK
N_PAIRS = (N_CHUNKS - 1) // 2
EB = 4
LANES = 16
NCB = D // LANES

ACC_BLOCK = 8000


def _tc_body(x_ref, wrbf_ref, wedge_ref, b_ref, idxj_ref, idxi_ref, rbf_ref,
             t_ref, idx_ref, acc_ref):
    wr = jnp.dot(wrbf_ref[...], wedge_ref[2 * D:3 * D, :],
                 preferred_element_type=jnp.float32)
    acc_ref[...] = jnp.dot(rbf_ref[...], wr,
                           preferred_element_type=jnp.float32)

    @pl.when(pl.program_id(0) == 0)
    def _():
        x = x_ref[...]
        t_ref[0:N_NODES, :] = jnp.dot(x, wedge_ref[0:D, :],
                                      preferred_element_type=jnp.float32)
        t_ref[N_NODES:2 * N_NODES, :] = (
            jnp.dot(x, wedge_ref[D:2 * D, :],
                    preferred_element_type=jnp.float32)
            + b_ref[...]
        )
        idx_ref[0] = idxj_ref[...]
        idx_ref[1] = idxi_ref[...] + N_NODES


def _tc_stage(x, W_rbf, W_edge, b_edge, idx_j, idx_i, rbf):
    full = lambda shape: pl.BlockSpec(shape, lambda i: tuple(0 for _ in shape))
    n_idx_rows = N_EDGES // D
    return pl.pallas_call(
        _tc_body,
        grid=(N_EDGES // ACC_BLOCK,),
        in_specs=[
            full((N_NODES, D)),
            full((NR, D)),
            full((3 * D, D)),
            full((1, D)),
            full((n_idx_rows, D)),
            full((n_idx_rows, D)),
            pl.BlockSpec((ACC_BLOCK, NR), lambda i: (i, 0)),
        ],
        out_specs=[
            full((2 * N_NODES, D)),
            full((2, n_idx_rows, D)),
            pl.BlockSpec((ACC_BLOCK, D), lambda i: (i, 0)),
        ],
        out_shape=[
            jax.ShapeDtypeStruct((2 * N_NODES, D), jnp.float32),
            jax.ShapeDtypeStruct((2, n_idx_rows, D), jnp.int32),
            jax.ShapeDtypeStruct((N_EDGES, D), jnp.float32),
        ],
    )(x, W_rbf, W_edge, b_edge.reshape(1, D),
      idx_j.reshape(n_idx_rows, D), idx_i.reshape(n_idx_rows, D), rbf)


def _edge_body(t_hbm, acc_hbm, idx_hbm, out_hbm,
               idx_v0, idx_v1, rows_j, rows_i, acc_v, out_v,
               sem_g0, sem_g1, sem_a0, sem_a1, sem_o0, sem_o1):
    wid = lax.axis_index("s") * NC + lax.axis_index("c")
    base_w = wid * EPW
    sem_g = (sem_g0, sem_g1)
    sem_a = (sem_a0, sem_a1)
    sem_o = (sem_o0, sem_o1)
    rows = ((rows_j.at[0], rows_i.at[0]), (rows_j.at[1], rows_i.at[1]))
    accb = (acc_v.at[0], acc_v.at[1])
    outb = (out_v.at[0], out_v.at[1])

    pltpu.sync_copy(idx_hbm.at[0, wid, 0, :], idx_v0)
    pltpu.sync_copy(idx_hbm.at[1, wid, 0, :], idx_v1)

    def issue_in(c, b):
        off = c * CHUNK
        pltpu.async_copy(t_hbm.at[idx_v0.at[pl.ds(off, CHUNK)]],
                         rows[b][0], sem_g[b])
        pltpu.async_copy(t_hbm.at[idx_v1.at[pl.ds(off, CHUNK)]],
                         rows[b][1], sem_g[b])
        pltpu.async_copy(acc_hbm.at[pl.ds(base_w + off, CHUNK)],
                         accb[b], sem_a[b])

    def wait_in(b):
        pltpu.make_async_copy(t_hbm.at[idx_v0.at[pl.ds(0, CHUNK)]],
                              rows[b][0], sem_g[b]).wait()
        pltpu.make_async_copy(t_hbm.at[idx_v1.at[pl.ds(0, CHUNK)]],
                              rows[b][1], sem_g[b]).wait()
        pltpu.make_async_copy(acc_hbm.at[pl.ds(0, CHUNK)],
                              accb[b], sem_a[b]).wait()

    def wait_out(b):
        pltpu.make_async_copy(outb[b], out_hbm.at[pl.ds(0, CHUNK)],
                              sem_o[b]).wait()

    def compute_store(c, b):
        rj, ri = rows[b]
        av, ov = accb[b], outb[b]

        def eb_body(i, _):
            e0 = i * EB
            for ep in range(EB):
                e = e0 + ep
                for cb in range(NCB):
                    sl = pl.ds(cb * LANES, LANES)
                    t = rj[e, sl] + ri[e, sl] + av[e, sl]
                    ov[e, sl] = t / (1.0 + jnp.exp(-t))
            return 0

        lax.fori_loop(0, CHUNK // EB, eb_body, 0)
        pltpu.async_copy(ov, out_hbm.at[pl.ds(base_w + c * CHUNK, CHUNK)],
                         sem_o[b])

    issue_in(0, 0)
    issue_in(1, 1)

    def pair_body(p, _):
        c0 = 2 * p
        for b in (0, 1):
            c = c0 + b
            wait_in(b)

            @pl.when(c >= 2)
            def _():
                wait_out(b)

            compute_store(c, b)

            @pl.when(c + 2 < N_CHUNKS)
            def _():
                issue_in(c + 2, b)

        return 0

    lax.fori_loop(0, N_PAIRS, pair_body, 0)
    wait_in(0)
    wait_out(0)
    compute_store(N_CHUNKS - 1, 0)
    wait_out(1)
    wait_out(0)


def _edge_kernel(t, acc, idx_cat):
    mesh = plsc.VectorSubcoreMesh(core_axis_name="c", subcore_axis_name="s")
    return pl.kernel(
        _edge_body,
        out_type=jax.ShapeDtypeStruct((N_EDGES, D), jnp.float32),
        mesh=mesh,
        scratch_types=[
            pltpu.VMEM((EPW,), jnp.int32),
            pltpu.VMEM((EPW,), jnp.int32),
            pltpu.VMEM((2, CHUNK, D), jnp.float32),
            pltpu.VMEM((2, CHUNK, D), jnp.float32),
            pltpu.VMEM((2, CHUNK, D), jnp.float32),
            pltpu.VMEM((2, CHUNK, D), jnp.float32),
        ] + [pltpu.SemaphoreType.DMA] * 6,
    )(t, acc, idx_cat)


def kernel(x, rbf, idx_i, idx_j, W_rbf, W_edge, b_edge):
    idx_i = idx_i.astype(jnp.int32)
    idx_j = idx_j.astype(jnp.int32)
    t, idx_cat, acc = _tc_stage(x, W_rbf, W_edge, b_edge, idx_j, idx_i, rbf)
    return _edge_kernel(t, acc, idx_cat.reshape(2, NW, 1, EPW))
